# baseline (device time: 52932 ns/iter reference)
import os

import jax
import jax.numpy as jnp
from jax import lax
from jax.experimental import pallas as pl
from jax.experimental.pallas import tpu as pltpu

_SKIP_COMM = os.environ.get("SKIP_COMM") == "1"
_SKIP_COMPUTE = os.environ.get("SKIP_COMPUTE") == "1"

N_DEV = 32
B, SQ, SKV, DH = 2, 256, 256, 64
HL = 4
DM = 512
HCOLS = HL * DH
ROWS = B * SQ
CH = ROWS // N_DEV
BLK = ROWS // 8


def _lindex(x, y, z):
    p = (y << 1) | (x ^ (y & 1))
    return (z << 3) | p


def kernel(x, Wq, K_ext, V_ext, Wo):
    i = lax.axis_index("i")
    Wq_l = lax.dynamic_slice(Wq, (0, i * HCOLS), (DM, HCOLS))
    Wo_l = lax.dynamic_slice(Wo, (i * HCOLS, 0), (HCOLS, DM))

    def body(x_ref, wq_ref, k_ref, v_ref, wo_ref, out_ref,
             acc, rbuf, zbuf, a_send, a_recv, b_send, b_recv,
             c_send, c_recv, d_send, d_recv):
        me = lax.axis_index("i")
        zc = me >> 3
        p = me & 7
        yb = p >> 1
        xb = (p & 1) ^ (yb & 1)
        zb = zc

        maskf = (
            jnp.abs(
                lax.broadcasted_iota(jnp.int32, (SQ, SKV), 0)
                - lax.broadcasted_iota(jnp.int32, (SQ, SKV), 1)
            )
            <= 128
        ).astype(jnp.float32)
        if _SKIP_COMPUTE:
            acc[:, :] = jnp.reshape(x_ref[:, :, :], (ROWS, DM))
        else:
            x2 = jnp.reshape(x_ref[:, :, :], (ROWS, DM))
            q2 = jnp.dot(x2, wq_ref[:, :], preferred_element_type=jnp.float32)
            ctx_rows = []
            for b in range(B):
                ctx_cols = []
                for h in range(HL):
                    q_h = q2[b * SQ:(b + 1) * SQ, h * DH:(h + 1) * DH]
                    k_h = k_ref[b, :, h, :]
                    v_h = v_ref[b, :, h, :]
                    s = lax.dot_general(
                        q_h, k_h, (((1,), (1,)), ((), ())),
                        preferred_element_type=jnp.float32,
                    ) * 0.125
                    e = jnp.exp(s) * maskf
                    denom = jnp.sum(e, axis=-1, keepdims=True)
                    ctx_cols.append(
                        jnp.dot(e, v_h, preferred_element_type=jnp.float32)
                        / denom
                    )
                ctx_rows.append(jnp.concatenate(ctx_cols, axis=1))
            ctx2 = jnp.concatenate(ctx_rows, axis=0)
            acc[:, :] = jnp.dot(
                ctx2, wo_ref[:, :], preferred_element_type=jnp.float32
            )

        if _SKIP_COMM:
            out_ref[0, :, :] = acc[0:SQ, :]
            out_ref[1, :, :] = acc[SQ:ROWS, :]
            return

        q = xb * 4 + yb
        blk = q * BLK
        sub = blk + zb * CH

        a_descs = []
        for dq in range(1, 8):
            qp = q ^ dq
            peer = _lindex(qp >> 2, qp & 3, zb)
            rdma = pltpu.make_async_remote_copy(
                src_ref=acc.at[pl.ds(qp * BLK, BLK)],
                dst_ref=rbuf.at[pl.ds((dq - 1) * BLK, BLK)],
                send_sem=a_send.at[dq - 1],
                recv_sem=a_recv.at[dq - 1],
                device_id=(peer,),
                device_id_type=pl.DeviceIdType.MESH,
            )
            rdma.start()
            a_descs.append(rdma)
        for rdma in a_descs:
            rdma.wait_recv()
        acc[pl.ds(blk, BLK), :] = acc[pl.ds(blk, BLK), :] + jnp.sum(
            jnp.reshape(rbuf[0:7 * BLK, :], (7, BLK, DM)), axis=0
        )

        b_descs = []
        for dz in range(1, 4):
            peer = _lindex(xb, yb, zb ^ dz)
            rdma = pltpu.make_async_remote_copy(
                src_ref=acc.at[pl.ds(blk + (zb ^ dz) * CH, CH)],
                dst_ref=zbuf.at[pl.ds((dz - 1) * CH, CH)],
                send_sem=b_send.at[dz - 1],
                recv_sem=b_recv.at[dz - 1],
                device_id=(peer,),
                device_id_type=pl.DeviceIdType.MESH,
            )
            rdma.start()
            b_descs.append(rdma)
        for rdma in b_descs:
            rdma.wait_recv()
        acc[pl.ds(sub, CH), :] = acc[pl.ds(sub, CH), :] + jnp.sum(
            jnp.reshape(zbuf[0:3 * CH, :], (3, CH, DM)), axis=0
        )

        c_descs = []
        for dz in range(1, 4):
            peer = _lindex(xb, yb, zb ^ dz)
            rdma = pltpu.make_async_remote_copy(
                src_ref=acc.at[pl.ds(sub, CH)],
                dst_ref=acc.at[pl.ds(sub, CH)],
                send_sem=c_send.at[dz - 1],
                recv_sem=c_recv.at[dz - 1],
                device_id=(peer,),
                device_id_type=pl.DeviceIdType.MESH,
            )
            rdma.start()
            c_descs.append(rdma)
        for rdma in c_descs:
            rdma.wait_recv()

        d_descs = []
        for dq in range(1, 8):
            qp = q ^ dq
            peer = _lindex(qp >> 2, qp & 3, zb)
            rdma = pltpu.make_async_remote_copy(
                src_ref=acc.at[pl.ds(blk, BLK)],
                dst_ref=acc.at[pl.ds(blk, BLK)],
                send_sem=d_send.at[dq - 1],
                recv_sem=d_recv.at[dq - 1],
                device_id=(peer,),
                device_id_type=pl.DeviceIdType.MESH,
            )
            rdma.start()
            d_descs.append(rdma)
        for rdma in d_descs:
            rdma.wait_recv()

        out_ref[0, :, :] = acc[0:SQ, :]
        out_ref[1, :, :] = acc[SQ:ROWS, :]

        for rdma in a_descs + b_descs + c_descs + d_descs:
            rdma.wait_send()

    return pl.pallas_call(
        body,
        out_shape=jax.ShapeDtypeStruct((B, SQ, DM), jnp.float32),
        in_specs=[pl.BlockSpec(memory_space=pltpu.VMEM)] * 5,
        out_specs=pl.BlockSpec(memory_space=pltpu.VMEM),
        scratch_shapes=[
            pltpu.VMEM((ROWS, DM), jnp.float32),
            pltpu.VMEM((7 * BLK, DM), jnp.float32),
            pltpu.VMEM((3 * CH, DM), jnp.float32),
            pltpu.SemaphoreType.DMA((7,)),
            pltpu.SemaphoreType.DMA((7,)),
            pltpu.SemaphoreType.DMA((3,)),
            pltpu.SemaphoreType.DMA((3,)),
            pltpu.SemaphoreType.DMA((3,)),
            pltpu.SemaphoreType.DMA((3,)),
            pltpu.SemaphoreType.DMA((7,)),
            pltpu.SemaphoreType.DMA((7,)),
        ],
    )(x, Wq_l, K_ext, V_ext, Wo_l)


# device time: 41851 ns/iter; 1.2648x vs baseline; 1.2648x over previous
import os

import jax
import jax.numpy as jnp
from jax import lax
from jax.experimental import pallas as pl
from jax.experimental.pallas import tpu as pltpu

_SKIP_COMM = os.environ.get("SKIP_COMM") == "1"
_SKIP_COMPUTE = os.environ.get("SKIP_COMPUTE") == "1"

N_DEV = 32
B, SQ, SKV, DH = 2, 256, 256, 64
HL = 4
DM = 512
HCOLS = HL * DH
ROWS = B * SQ
CH = ROWS // N_DEV
BLK = ROWS // 8


def _lindex(x, y, z):
    p = (y << 1) | (x ^ (y & 1))
    return (z << 3) | p


def kernel(x, Wq, K_ext, V_ext, Wo):
    i = lax.axis_index("i")
    Wq_l = lax.dynamic_slice(Wq, (0, i * HCOLS), (DM, HCOLS))
    Wo_l = lax.dynamic_slice(Wo, (i * HCOLS, 0), (HCOLS, DM))

    def body(x_ref, wq_ref, k_ref, v_ref, wo_ref, out_ref,
             acc, acc16, rbuf, zbuf, a_send, a_recv, b_send, b_recv,
             c_send, c_recv, d_send, d_recv):
        me = lax.axis_index("i")
        zc = me >> 3
        p = me & 7
        yb = p >> 1
        xb = (p & 1) ^ (yb & 1)
        zb = zc

        maskf = (
            jnp.abs(
                lax.broadcasted_iota(jnp.int32, (SQ, SKV), 0)
                - lax.broadcasted_iota(jnp.int32, (SQ, SKV), 1)
            )
            <= 128
        ).astype(jnp.float32)
        if _SKIP_COMPUTE:
            acc[:, :] = jnp.reshape(x_ref[:, :, :], (ROWS, DM))
        else:
            x2 = jnp.reshape(x_ref[:, :, :], (ROWS, DM))
            q2 = jnp.dot(x2, wq_ref[:, :], preferred_element_type=jnp.float32)
            ctx_rows = []
            for b in range(B):
                ctx_cols = []
                for h in range(HL):
                    q_h = q2[b * SQ:(b + 1) * SQ, h * DH:(h + 1) * DH]
                    k_h = k_ref[b, :, h, :]
                    v_h = v_ref[b, :, h, :]
                    s = lax.dot_general(
                        q_h, k_h, (((1,), (1,)), ((), ())),
                        preferred_element_type=jnp.float32,
                    ) * 0.125
                    e = jnp.exp(s) * maskf
                    denom = jnp.sum(e, axis=-1, keepdims=True)
                    ctx_cols.append(
                        jnp.dot(e, v_h, preferred_element_type=jnp.float32)
                        / denom
                    )
                ctx_rows.append(jnp.concatenate(ctx_cols, axis=1))
            ctx2 = jnp.concatenate(ctx_rows, axis=0)
            acc[:, :] = jnp.dot(
                ctx2, wo_ref[:, :], preferred_element_type=jnp.float32
            )

        if _SKIP_COMM:
            out_ref[0, :, :] = acc[0:SQ, :]
            out_ref[1, :, :] = acc[SQ:ROWS, :]
            return

        q = xb * 4 + yb
        blk = q * BLK
        sub = blk + zb * CH

        acc16[:, :] = acc[:, :].astype(jnp.bfloat16)

        a_descs = []
        for dq in range(1, 8):
            qp = q ^ dq
            peer = _lindex(qp >> 2, qp & 3, zb)
            rdma = pltpu.make_async_remote_copy(
                src_ref=acc16.at[pl.ds(qp * BLK, BLK)],
                dst_ref=rbuf.at[pl.ds((dq - 1) * BLK, BLK)],
                send_sem=a_send.at[dq - 1],
                recv_sem=a_recv.at[dq - 1],
                device_id=(peer,),
                device_id_type=pl.DeviceIdType.MESH,
            )
            rdma.start()
            a_descs.append(rdma)
        for rdma in a_descs:
            rdma.wait_recv()
        blk_sum = acc[pl.ds(blk, BLK), :] + jnp.sum(
            jnp.reshape(
                rbuf[0:7 * BLK, :].astype(jnp.float32), (7, BLK, DM)
            ),
            axis=0,
        )
        acc[pl.ds(blk, BLK), :] = blk_sum
        acc16[pl.ds(blk, BLK), :] = blk_sum.astype(jnp.bfloat16)

        b_descs = []
        for dz in range(1, 4):
            peer = _lindex(xb, yb, zb ^ dz)
            rdma = pltpu.make_async_remote_copy(
                src_ref=acc16.at[pl.ds(blk + (zb ^ dz) * CH, CH)],
                dst_ref=zbuf.at[pl.ds((dz - 1) * CH, CH)],
                send_sem=b_send.at[dz - 1],
                recv_sem=b_recv.at[dz - 1],
                device_id=(peer,),
                device_id_type=pl.DeviceIdType.MESH,
            )
            rdma.start()
            b_descs.append(rdma)
        for rdma in b_descs:
            rdma.wait_recv()
        sub_sum = acc[pl.ds(sub, CH), :] + jnp.sum(
            jnp.reshape(
                zbuf[0:3 * CH, :].astype(jnp.float32), (3, CH, DM)
            ),
            axis=0,
        )
        acc16[pl.ds(sub, CH), :] = sub_sum.astype(jnp.bfloat16)

        c_descs = []
        for dz in range(1, 4):
            peer = _lindex(xb, yb, zb ^ dz)
            rdma = pltpu.make_async_remote_copy(
                src_ref=acc16.at[pl.ds(sub, CH)],
                dst_ref=acc16.at[pl.ds(sub, CH)],
                send_sem=c_send.at[dz - 1],
                recv_sem=c_recv.at[dz - 1],
                device_id=(peer,),
                device_id_type=pl.DeviceIdType.MESH,
            )
            rdma.start()
            c_descs.append(rdma)
        for rdma in c_descs:
            rdma.wait_recv()

        d_descs = []
        for dq in range(1, 8):
            qp = q ^ dq
            peer = _lindex(qp >> 2, qp & 3, zb)
            rdma = pltpu.make_async_remote_copy(
                src_ref=acc16.at[pl.ds(blk, BLK)],
                dst_ref=acc16.at[pl.ds(blk, BLK)],
                send_sem=d_send.at[dq - 1],
                recv_sem=d_recv.at[dq - 1],
                device_id=(peer,),
                device_id_type=pl.DeviceIdType.MESH,
            )
            rdma.start()
            d_descs.append(rdma)
        for rdma in d_descs:
            rdma.wait_recv()

        out_ref[0, :, :] = acc16[0:SQ, :].astype(jnp.float32)
        out_ref[1, :, :] = acc16[SQ:ROWS, :].astype(jnp.float32)

        for rdma in a_descs + b_descs + c_descs + d_descs:
            rdma.wait_send()

    return pl.pallas_call(
        body,
        out_shape=jax.ShapeDtypeStruct((B, SQ, DM), jnp.float32),
        in_specs=[pl.BlockSpec(memory_space=pltpu.VMEM)] * 5,
        out_specs=pl.BlockSpec(memory_space=pltpu.VMEM),
        scratch_shapes=[
            pltpu.VMEM((ROWS, DM), jnp.float32),
            pltpu.VMEM((ROWS, DM), jnp.bfloat16),
            pltpu.VMEM((7 * BLK, DM), jnp.bfloat16),
            pltpu.VMEM((3 * CH, DM), jnp.bfloat16),
            pltpu.SemaphoreType.DMA((7,)),
            pltpu.SemaphoreType.DMA((7,)),
            pltpu.SemaphoreType.DMA((3,)),
            pltpu.SemaphoreType.DMA((3,)),
            pltpu.SemaphoreType.DMA((3,)),
            pltpu.SemaphoreType.DMA((3,)),
            pltpu.SemaphoreType.DMA((7,)),
            pltpu.SemaphoreType.DMA((7,)),
        ],
    )(x, Wq_l, K_ext, V_ext, Wo_l)


# device time: 22000 ns/iter; 2.4060x vs baseline; 1.9023x over previous
import os

import jax
import jax.numpy as jnp
from jax import lax
from jax.experimental import pallas as pl
from jax.experimental.pallas import tpu as pltpu

_SKIP_COMM = os.environ.get("SKIP_COMM") == "1"
_SKIP_COMPUTE = os.environ.get("SKIP_COMPUTE") == "1"
_PHASES = os.environ.get("PHASES", "ABCD")

N_DEV = 32
B, SQ, SKV, DH = 2, 256, 256, 64
HL = 4
DM = 512
HCOLS = HL * DH
ROWS = B * SQ
CH = ROWS // N_DEV
BLK = ROWS // 8


def _lindex(x, y, z):
    p = (y << 1) | (x ^ (y & 1))
    return (z << 3) | p


def kernel(x, Wq, K_ext, V_ext, Wo):
    i = lax.axis_index("i")
    Wq_l = lax.dynamic_slice(Wq, (0, i * HCOLS), (DM, HCOLS))
    Wo_l = lax.dynamic_slice(Wo, (i * HCOLS, 0), (HCOLS, DM))

    def body(x_ref, wq_ref, k_ref, v_ref, wo_ref, out_ref,
             acc, acc16, rbuf, zbuf, a_send, a_recv, b_send, b_recv,
             c_send, c_recv, d_send, d_recv):
        me = lax.axis_index("i")
        zc = me >> 3
        p = me & 7
        yb = p >> 1
        xb = (p & 1) ^ (yb & 1)
        zb = zc

        maskf = (
            jnp.abs(
                lax.broadcasted_iota(jnp.int32, (SQ, SKV), 0)
                - lax.broadcasted_iota(jnp.int32, (SQ, SKV), 1)
            )
            <= 128
        ).astype(jnp.float32)
        if _SKIP_COMPUTE:
            acc[:, :] = jnp.reshape(x_ref[:, :, :], (ROWS, DM))
        else:
            x2 = jnp.reshape(x_ref[:, :, :], (ROWS, DM))
            q2 = jnp.dot(x2, wq_ref[:, :], preferred_element_type=jnp.float32)
            ctx_rows = []
            for b in range(B):
                ctx_cols = []
                for h in range(HL):
                    q_h = q2[b * SQ:(b + 1) * SQ, h * DH:(h + 1) * DH]
                    k_h = k_ref[b, :, h, :]
                    v_h = v_ref[b, :, h, :]
                    s = lax.dot_general(
                        q_h, k_h, (((1,), (1,)), ((), ())),
                        preferred_element_type=jnp.float32,
                    ) * 0.125
                    e = jnp.exp(s) * maskf
                    denom = jnp.sum(e, axis=-1, keepdims=True)
                    ctx_cols.append(
                        jnp.dot(e, v_h, preferred_element_type=jnp.float32)
                        / denom
                    )
                ctx_rows.append(jnp.concatenate(ctx_cols, axis=1))
            ctx2 = jnp.concatenate(ctx_rows, axis=0)
            acc[:, :] = jnp.dot(
                ctx2, wo_ref[:, :], preferred_element_type=jnp.float32
            )

        if _SKIP_COMM:
            out_ref[0, :, :] = acc[0:SQ, :]
            out_ref[1, :, :] = acc[SQ:ROWS, :]
            return

        q = xb * 4 + yb
        blk = q * BLK
        sub = blk + zb * CH

        acc16[:, :] = acc[:, :].astype(jnp.bfloat16)

        a_descs = []
        for dq in range(1, 8) if 'A' in _PHASES else []:
            qp = q ^ dq
            peer = _lindex(qp >> 2, qp & 3, zb)
            rdma = pltpu.make_async_remote_copy(
                src_ref=acc16.at[pl.ds(qp * BLK, BLK)],
                dst_ref=rbuf.at[pl.ds((dq - 1) * BLK, BLK)],
                send_sem=a_send.at[dq - 1],
                recv_sem=a_recv.at[dq - 1],
                device_id=(peer,),
                device_id_type=pl.DeviceIdType.MESH,
            )
            rdma.start()
            a_descs.append(rdma)
        for rdma in a_descs:
            rdma.wait_recv()
        blk_sum = acc[pl.ds(blk, BLK), :] + jnp.sum(
            jnp.reshape(
                rbuf[0:7 * BLK, :].astype(jnp.float32), (7, BLK, DM)
            ),
            axis=0,
        )
        acc[pl.ds(blk, BLK), :] = blk_sum
        acc16[pl.ds(blk, BLK), :] = blk_sum.astype(jnp.bfloat16)

        b_descs = []
        for dz in range(1, 4) if 'B' in _PHASES else []:
            peer = _lindex(xb, yb, zb ^ dz)
            rdma = pltpu.make_async_remote_copy(
                src_ref=acc16.at[pl.ds(blk + (zb ^ dz) * CH, CH)],
                dst_ref=zbuf.at[pl.ds((dz - 1) * CH, CH)],
                send_sem=b_send.at[dz - 1],
                recv_sem=b_recv.at[dz - 1],
                device_id=(peer,),
                device_id_type=pl.DeviceIdType.MESH,
            )
            rdma.start()
            b_descs.append(rdma)
        for rdma in b_descs:
            rdma.wait_recv()
        sub_sum = acc[pl.ds(sub, CH), :] + jnp.sum(
            jnp.reshape(
                zbuf[0:3 * CH, :].astype(jnp.float32), (3, CH, DM)
            ),
            axis=0,
        )
        acc16[pl.ds(sub, CH), :] = sub_sum.astype(jnp.bfloat16)

        c_descs = []
        for dz in range(1, 4) if 'C' in _PHASES else []:
            peer = _lindex(xb, yb, zb ^ dz)
            rdma = pltpu.make_async_remote_copy(
                src_ref=acc16.at[pl.ds(sub, CH)],
                dst_ref=acc16.at[pl.ds(sub, CH)],
                send_sem=c_send.at[dz - 1],
                recv_sem=c_recv.at[dz - 1],
                device_id=(peer,),
                device_id_type=pl.DeviceIdType.MESH,
            )
            rdma.start()
            c_descs.append(rdma)
        for rdma in c_descs:
            rdma.wait_recv()

        d_descs = []
        for dq in range(1, 8) if 'D' in _PHASES else []:
            qp = q ^ dq
            peer = _lindex(qp >> 2, qp & 3, zb)
            rdma = pltpu.make_async_remote_copy(
                src_ref=acc16.at[pl.ds(blk, BLK)],
                dst_ref=acc16.at[pl.ds(blk, BLK)],
                send_sem=d_send.at[dq - 1],
                recv_sem=d_recv.at[dq - 1],
                device_id=(peer,),
                device_id_type=pl.DeviceIdType.MESH,
            )
            rdma.start()
            d_descs.append(rdma)
        for rdma in d_descs:
            rdma.wait_recv()

        out_ref[0, :, :] = acc16[0:SQ, :].astype(jnp.float32)
        out_ref[1, :, :] = acc16[SQ:ROWS, :].astype(jnp.float32)

        for rdma in a_descs + b_descs + c_descs + d_descs:
            rdma.wait_send()

    return pl.pallas_call(
        body,
        out_shape=jax.ShapeDtypeStruct((B, SQ, DM), jnp.float32),
        in_specs=[pl.BlockSpec(memory_space=pltpu.VMEM)] * 5,
        out_specs=pl.BlockSpec(memory_space=pltpu.VMEM),
        scratch_shapes=[
            pltpu.VMEM((ROWS, DM), jnp.float32),
            pltpu.VMEM((ROWS, DM), jnp.bfloat16),
            pltpu.VMEM((7 * BLK, DM), jnp.bfloat16),
            pltpu.VMEM((3 * CH, DM), jnp.bfloat16),
            pltpu.SemaphoreType.DMA((7,)),
            pltpu.SemaphoreType.DMA((7,)),
            pltpu.SemaphoreType.DMA((3,)),
            pltpu.SemaphoreType.DMA((3,)),
            pltpu.SemaphoreType.DMA((3,)),
            pltpu.SemaphoreType.DMA((3,)),
            pltpu.SemaphoreType.DMA((7,)),
            pltpu.SemaphoreType.DMA((7,)),
        ],
    )(x, Wq_l, K_ext, V_ext, Wo_l)
